# baseline (speedup 1.0000x reference)
import jax
import jax.numpy as jnp
from jax import lax
from jax.experimental import pallas as pl
from jax.experimental.pallas import tpu as pltpu
from jax.experimental.pallas import tpu_sc as plsc

N_FIELDS = 22
VOCAB = 12
EMB_DIM = 64
BATCH = 16384

N_PAIRS = N_FIELDS // 2
PAIR_DIM = 2 * EMB_DIM
PAIR_VOCAB = VOCAB * VOCAB
TOTAL_PROWS = BATCH * N_PAIRS
NUM_WORKERS = 32
PROWS_PER_WORKER = TOTAL_PROWS // NUM_WORKERS
ROWS_PER_WORKER = BATCH // NUM_WORKERS
STRIPE = 88
NCHUNKS = PROWS_PER_WORKER // STRIPE
LANES = 16
NBUF = 8
NITER = NCHUNKS // NBUF


def _body(xp_hbm, ptable_hbm, out_hbm, xbuf, idxbuf, rowsbuf, *sems):
    gsem = sems[:NBUF]
    wsem = sems[NBUF:]
    wid = lax.axis_index("s") * 2 + lax.axis_index("c")
    lane = lax.iota(jnp.int32, LANES)
    pbase = wid * PROWS_PER_WORKER

    pltpu.sync_copy(xp_hbm.at[pl.ds(pbase, PROWS_PER_WORKER)], xbuf)

    EV = []
    TV = []
    for j in range(11):
        q = j * LANES + lane
        m = q % STRIPE
        t = m // 8
        i = m % 8
        cc = q // STRIPE
        EV.append((8 * cc + i) * N_PAIRS + t)
        TV.append(t * PAIR_VOCAB)

    def id_body(r, c):
        base = jnp.full((LANES,), 176 * r, jnp.int32)
        for j in range(11):
            v = plsc.load_gather(xbuf, [base + EV[j]])
            ev = v & 0xFFFF
            od = v >> 16
            idxbuf[pl.ds(r * 176 + j * LANES, LANES)] = ev * VOCAB + od + TV[j]
        return c

    lax.fori_loop(0, PROWS_PER_WORKER // 176, id_body, 0)

    def g_start(slot, g):
        pltpu.async_copy(
            ptable_hbm.at[idxbuf.at[pl.ds(g * STRIPE, STRIPE)]],
            rowsbuf.at[slot], gsem[slot])

    def g_wait(slot, g):
        pltpu.make_async_copy(
            ptable_hbm.at[idxbuf.at[pl.ds(g * STRIPE, STRIPE)]],
            rowsbuf.at[slot], gsem[slot]).wait()

    def w_start(slot, g):
        pltpu.async_copy(
            rowsbuf.at[slot],
            out_hbm.at[pl.ds(pbase + g * STRIPE, STRIPE)], wsem[slot])

    def w_wait(slot, g):
        pltpu.make_async_copy(
            rowsbuf.at[slot],
            out_hbm.at[pl.ds(pbase + g * STRIPE, STRIPE)], wsem[slot]).wait()

    for b in range(NBUF):
        g_start(b, b)

    def block(k, c):
        for b in range(NBUF):
            g_wait(b, k * NBUF + b)
            w_start(b, k * NBUF + b)

        @pl.when(k < NITER - 1)
        def _():
            for b in range(NBUF):
                w_wait(b, k * NBUF + b)
                g_start(b, (k + 1) * NBUF + b)

        return c

    lax.fori_loop(0, NITER, block, 0)

    for b in range(NBUF):
        w_wait(b, (NITER - 1) * NBUF + b)


@jax.jit
def _gather(xflat, ptable):
    mesh = plsc.VectorSubcoreMesh(core_axis_name="c", subcore_axis_name="s")
    return pl.kernel(
        _body,
        out_type=jax.ShapeDtypeStruct((TOTAL_PROWS, PAIR_DIM), jnp.float32),
        mesh=mesh,
        scratch_types=[
            pltpu.VMEM((PROWS_PER_WORKER,), jnp.int32),
            pltpu.VMEM((PROWS_PER_WORKER,), jnp.int32),
            pltpu.VMEM((NBUF, STRIPE, PAIR_DIM), jnp.float32),
        ] + [pltpu.SemaphoreType.DMA] * (2 * NBUF),
        compiler_params=pltpu.CompilerParams(
            use_tc_tiling_on_sc=False, needs_layout_passes=False),
    )(xflat, ptable)


def kernel(x, W):
    xi = x.astype(jnp.int32)
    xp = (xi[:, 0::2] | (xi[:, 1::2] << 16)).reshape(-1)
    We = jnp.broadcast_to(W[0::2][:, :, None, :], (N_PAIRS, VOCAB, VOCAB, EMB_DIM))
    Wo = jnp.broadcast_to(W[1::2][:, None, :, :], (N_PAIRS, VOCAB, VOCAB, EMB_DIM))
    ptable = jnp.concatenate([We, Wo], axis=-1).reshape(N_PAIRS * PAIR_VOCAB, PAIR_DIM)
    out = _gather(xp, ptable)
    return (out.reshape(BATCH // 8, N_PAIRS, 8, PAIR_DIM)
            .transpose(0, 2, 1, 3)
            .reshape(BATCH, N_FIELDS * EMB_DIM))


# speedup vs baseline: 1.0674x; 1.0674x over previous
import jax
import jax.numpy as jnp
from jax import lax
from jax.experimental import pallas as pl
from jax.experimental.pallas import tpu as pltpu
from jax.experimental.pallas import tpu_sc as plsc

N_FIELDS = 22
VOCAB = 12
EMB_DIM = 64
BATCH = 16384

N_PAIRS = N_FIELDS // 2
PAIR_DIM = 2 * EMB_DIM
PAIR_VOCAB = VOCAB * VOCAB
TOTAL_PROWS = BATCH * N_PAIRS
NUM_WORKERS = 32
PROWS_PER_WORKER = TOTAL_PROWS // NUM_WORKERS
ROWS_PER_WORKER = BATCH // NUM_WORKERS
STRIPE = 88
NCHUNKS = PROWS_PER_WORKER // STRIPE
LANES = 16
NBUF = 8
NITER = NCHUNKS // NBUF


def _body(xflat_hbm, ptable_hbm, out_hbm, xbuf, idxbuf, rowsbuf, *sems):
    gsem = sems[:NBUF]
    wsem = sems[NBUF:]
    wid = lax.axis_index("s") * 2 + lax.axis_index("c")
    lane = lax.iota(jnp.int32, LANES)
    pbase = wid * PROWS_PER_WORKER
    row0 = wid * ROWS_PER_WORKER

    pltpu.sync_copy(
        xflat_hbm.at[pl.ds(row0 * N_FIELDS, ROWS_PER_WORKER * N_FIELDS)],
        xbuf)

    EV = []
    TV = []
    for j in range(11):
        q = j * LANES + lane
        m = q % STRIPE
        t = m // 8
        i = m % 8
        cc = q // STRIPE
        EV.append((8 * cc + i) * N_FIELDS + 2 * t)
        TV.append(t * PAIR_VOCAB)

    def id_body(r, c):
        base = jnp.full((LANES,), 352 * r, jnp.int32)
        for j in range(11):
            ev = plsc.load_gather(xbuf, [base + EV[j]])
            od = plsc.load_gather(xbuf, [base + (EV[j] + 1)])
            idxbuf[pl.ds(r * 176 + j * LANES, LANES)] = ev * VOCAB + od + TV[j]
        return c

    lax.fori_loop(0, PROWS_PER_WORKER // 176, id_body, 0)

    def g_start(slot, g):
        pltpu.async_copy(
            ptable_hbm.at[idxbuf.at[pl.ds(g * STRIPE, STRIPE)]],
            rowsbuf.at[slot], gsem[slot])

    def g_wait(slot, g):
        pltpu.make_async_copy(
            ptable_hbm.at[idxbuf.at[pl.ds(g * STRIPE, STRIPE)]],
            rowsbuf.at[slot], gsem[slot]).wait()

    def w_start(slot, g):
        pltpu.async_copy(
            rowsbuf.at[slot],
            out_hbm.at[pl.ds(pbase + g * STRIPE, STRIPE)], wsem[slot])

    def w_wait(slot, g):
        pltpu.make_async_copy(
            rowsbuf.at[slot],
            out_hbm.at[pl.ds(pbase + g * STRIPE, STRIPE)], wsem[slot]).wait()

    for b in range(NBUF):
        g_start(b, b)

    def block(k, c):
        for b in range(NBUF):
            g_wait(b, k * NBUF + b)
            w_start(b, k * NBUF + b)

        @pl.when(k < NITER - 1)
        def _():
            for b in range(NBUF):
                w_wait(b, k * NBUF + b)
                g_start(b, (k + 1) * NBUF + b)

        return c

    lax.fori_loop(0, NITER, block, 0)

    for b in range(NBUF):
        w_wait(b, (NITER - 1) * NBUF + b)


@jax.jit
def _gather(xflat, ptable):
    mesh = plsc.VectorSubcoreMesh(core_axis_name="c", subcore_axis_name="s")
    return pl.kernel(
        _body,
        out_type=jax.ShapeDtypeStruct((TOTAL_PROWS, PAIR_DIM), jnp.float32),
        mesh=mesh,
        scratch_types=[
            pltpu.VMEM((ROWS_PER_WORKER * N_FIELDS,), jnp.int32),
            pltpu.VMEM((PROWS_PER_WORKER,), jnp.int32),
            pltpu.VMEM((NBUF, STRIPE, PAIR_DIM), jnp.float32),
        ] + [pltpu.SemaphoreType.DMA] * (2 * NBUF),
        compiler_params=pltpu.CompilerParams(
            use_tc_tiling_on_sc=False, needs_layout_passes=False),
    )(xflat, ptable)


def kernel(x, W):
    xflat = x.reshape(-1).astype(jnp.int32)
    We = jnp.broadcast_to(W[0::2][:, :, None, :], (N_PAIRS, VOCAB, VOCAB, EMB_DIM))
    Wo = jnp.broadcast_to(W[1::2][:, None, :, :], (N_PAIRS, VOCAB, VOCAB, EMB_DIM))
    ptable = jnp.concatenate([We, Wo], axis=-1).reshape(N_PAIRS * PAIR_VOCAB, PAIR_DIM)
    out = _gather(xflat, ptable)
    return (out.reshape(BATCH // 8, N_PAIRS, 8, PAIR_DIM)
            .transpose(0, 2, 1, 3)
            .reshape(BATCH, N_FIELDS * EMB_DIM))


# speedup vs baseline: 1.1063x; 1.0364x over previous
import jax
import jax.numpy as jnp
from jax import lax
from jax.experimental import pallas as pl
from jax.experimental.pallas import tpu as pltpu
from jax.experimental.pallas import tpu_sc as plsc

N_FIELDS = 22
VOCAB = 12
EMB_DIM = 64
BATCH = 16384

N_PAIRS = N_FIELDS // 2
PAIR_DIM = 2 * EMB_DIM
PAIR_VOCAB = VOCAB * VOCAB
TOTAL_PROWS = BATCH * N_PAIRS
NUM_WORKERS = 32
PROWS_PER_WORKER = TOTAL_PROWS // NUM_WORKERS
ROWS_PER_WORKER = BATCH // NUM_WORKERS
STRIPE = 88
NCHUNKS = PROWS_PER_WORKER // STRIPE
LANES = 16
NBUF = 8
NITER = NCHUNKS // NBUF


def _body(xflat_hbm, ptable_hbm, out_hbm, xbuf, idxbuf, rowsbuf, *sems):
    gsem = sems[:NBUF]
    wsem = sems[NBUF:]
    wid = lax.axis_index("s") * 2 + lax.axis_index("c")
    lane = lax.iota(jnp.int32, LANES)
    pbase = wid * PROWS_PER_WORKER
    row0 = wid * ROWS_PER_WORKER

    pltpu.sync_copy(
        xflat_hbm.at[pl.ds(row0 * N_FIELDS, ROWS_PER_WORKER * N_FIELDS)],
        xbuf)

    TV = []
    PK = []
    for j in range(11):
        q = j * LANES + lane
        t = q % N_PAIRS
        b = q // N_PAIRS
        TV.append(t * PAIR_VOCAB)
        PK.append((b // 8) * STRIPE + t * 8 + (b % 8))

    def id_body(r, c):
        for j in range(11):
            v = plsc.bitcast(xbuf[pl.ds((r * 11 + j) * 2 * LANES, 2 * LANES)],
                             jnp.int32)
            ev = v & 0xFFFF
            od = v >> 16
            plsc.store_scatter(
                idxbuf,
                [jnp.full((LANES,), 176 * r, jnp.int32) + PK[j]],
                ev * VOCAB + od + TV[j])
        return c

    lax.fori_loop(0, PROWS_PER_WORKER // 176, id_body, 0)

    def g_start(slot, g):
        pltpu.async_copy(
            ptable_hbm.at[idxbuf.at[pl.ds(g * STRIPE, STRIPE)]],
            rowsbuf.at[slot], gsem[slot])

    def g_wait(slot, g):
        pltpu.make_async_copy(
            ptable_hbm.at[idxbuf.at[pl.ds(g * STRIPE, STRIPE)]],
            rowsbuf.at[slot], gsem[slot]).wait()

    def w_start(slot, g):
        pltpu.async_copy(
            rowsbuf.at[slot],
            out_hbm.at[pl.ds(pbase + g * STRIPE, STRIPE)], wsem[slot])

    def w_wait(slot, g):
        pltpu.make_async_copy(
            rowsbuf.at[slot],
            out_hbm.at[pl.ds(pbase + g * STRIPE, STRIPE)], wsem[slot]).wait()

    for b in range(NBUF):
        g_start(b, b)

    def block(k, c):
        for b in range(NBUF):
            g_wait(b, k * NBUF + b)
            w_start(b, k * NBUF + b)

        @pl.when(k < NITER - 1)
        def _():
            for b in range(NBUF):
                w_wait(b, k * NBUF + b)
                g_start(b, (k + 1) * NBUF + b)

        return c

    lax.fori_loop(0, NITER, block, 0)

    for b in range(NBUF):
        w_wait(b, (NITER - 1) * NBUF + b)


@jax.jit
def _gather(xflat, ptable):
    mesh = plsc.VectorSubcoreMesh(core_axis_name="c", subcore_axis_name="s")
    return pl.kernel(
        _body,
        out_type=jax.ShapeDtypeStruct((TOTAL_PROWS, PAIR_DIM), jnp.float32),
        mesh=mesh,
        scratch_types=[
            pltpu.VMEM((ROWS_PER_WORKER * N_FIELDS,), jnp.int16),
            pltpu.VMEM((PROWS_PER_WORKER,), jnp.int32),
            pltpu.VMEM((NBUF, STRIPE, PAIR_DIM), jnp.float32),
        ] + [pltpu.SemaphoreType.DMA] * (2 * NBUF),
        compiler_params=pltpu.CompilerParams(
            use_tc_tiling_on_sc=False, needs_layout_passes=False),
    )(xflat, ptable)


def kernel(x, W):
    xflat = x.astype(jnp.int16).reshape(-1)
    We = jnp.broadcast_to(W[0::2][:, :, None, :], (N_PAIRS, VOCAB, VOCAB, EMB_DIM))
    Wo = jnp.broadcast_to(W[1::2][:, None, :, :], (N_PAIRS, VOCAB, VOCAB, EMB_DIM))
    ptable = jnp.concatenate([We, Wo], axis=-1).reshape(N_PAIRS * PAIR_VOCAB, PAIR_DIM)
    out = _gather(xflat, ptable)
    return (out.reshape(BATCH // 8, N_PAIRS, 8, PAIR_DIM)
            .transpose(0, 2, 1, 3)
            .reshape(BATCH, N_FIELDS * EMB_DIM))
